# baseline (speedup 1.0000x reference)
import functools

import jax
import jax.numpy as jnp
from jax import lax
from jax.experimental import pallas as pl
from jax.experimental.pallas import tpu as pltpu
from jax.experimental.pallas import tpu_sc as plsc

N = 10000
E = 320000
D = 128
B = 64
C = 10

NC = 2
NS = 16
NW = NC * NS
K = 80
Q = 40
EP = 481280
EPW = EP // NW
NCHUNK = EPW // K
NG = 4
GC = NCHUNK // NG
N_PAD = 10240
RPT = N_PAD // NS

BR = 2000
NBLK = N // BR


def _leaky(x):
    return jnp.where(x >= 0, x, 0.1 * x)


@functools.cache
def _get_sc_scatter():
    mesh = plsc.VectorSubcoreMesh(core_axis_name="c", subcore_axis_name="s")
    return pl.kernel(
        _sc_scatter_body,
        mesh=mesh,
        out_type=jax.ShapeDtypeStruct((2, N_PAD, D), jnp.float32),
        scratch_types=[
            pltpu.VMEM((GC, K), jnp.int32),
            pltpu.VMEM((GC, K), jnp.int32),
            pltpu.VMEM((K, D), jnp.float32),
            pltpu.VMEM_SHARED((N_PAD, D), jnp.float32),
            pltpu.SemaphoreType.DMA,
        ],
    )


def _sc_scatter_body(h_hbm, src_hbm, dst_hbm, zeros_hbm, out_hbm,
                     src_v, dst_v, rows_v, agg_sh, sem):
    cid = lax.axis_index("c")
    sid = lax.axis_index("s")
    wid = sid * NC + cid

    pltpu.sync_copy(zeros_hbm.at[pl.ds(sid * RPT, RPT)],
                    agg_sh.at[pl.ds(sid * RPT, RPT)])
    plsc.subcore_barrier()

    for g in range(NG):
        pltpu.sync_copy(src_hbm.at[wid, g], src_v)
        pltpu.sync_copy(dst_hbm.at[wid, g], dst_v)

        def body(c, _):
            pltpu.async_copy(h_hbm.at[src_v.at[c]], rows_v, sem).wait()
            pltpu.sync_copy(rows_v, agg_sh.at[dst_v.at[c]], add=True)
            return _

        lax.fori_loop(0, GC, body, None)
    plsc.subcore_barrier()

    pltpu.sync_copy(
        agg_sh.at[pl.ds(sid * RPT, RPT)],
        out_hbm.at[cid, pl.ds(sid * RPT, RPT)])


def _tc_layer_body(h_ref, p0_ref, p1_ref, wa_ref, ba_ref, wb_ref, bb_ref,
                   m2_ref, stats_ref):
    i = pl.program_id(0)
    m = h_ref[...] + (p0_ref[0] + p1_ref[0])
    m = _leaky(jnp.dot(m, wa_ref[...], preferred_element_type=jnp.float32,
                       precision=lax.Precision.DEFAULT) + ba_ref[...])
    m2 = _leaky(jnp.dot(m, wb_ref[...], preferred_element_type=jnp.float32,
                        precision=lax.Precision.DEFAULT) + bb_ref[...])
    m2_ref[...] = m2
    sm = jnp.sum(m2, axis=0, keepdims=True)
    upd = jnp.concatenate([sm, jnp.zeros((7, D), jnp.float32)], axis=0)

    @pl.when(i == 0)
    def _():
        stats_ref[...] = jnp.zeros_like(stats_ref)

    stats_ref[...] += upd


def _tc_layer(h, p, wa, ba, wb, bb):
    return pl.pallas_call(
        _tc_layer_body,
        grid=(NBLK,),
        in_specs=[
            pl.BlockSpec((BR, D), lambda i: (i, 0)),
            pl.BlockSpec((1, BR, D), lambda i: (0, i, 0)),
            pl.BlockSpec((1, BR, D), lambda i: (1, i, 0)),
            pl.BlockSpec((D, D), lambda i: (0, 0)),
            pl.BlockSpec((1, D), lambda i: (0, 0)),
            pl.BlockSpec((D, D), lambda i: (0, 0)),
            pl.BlockSpec((1, D), lambda i: (0, 0)),
        ],
        out_specs=[
            pl.BlockSpec((BR, D), lambda i: (i, 0)),
            pl.BlockSpec((8, D), lambda i: (0, 0)),
        ],
        out_shape=[
            jax.ShapeDtypeStruct((N, D), jnp.float32),
            jax.ShapeDtypeStruct((8, D), jnp.float32),
        ],
    )(h, p, p, wa, ba, wb, bb)


def _tc_var_body(m2_ref, stats_ref, sq_ref):
    i = pl.program_id(0)
    mean = stats_ref[0:1, :] / N
    dd = m2_ref[...] - mean
    v = jnp.sum(dd * dd, axis=0, keepdims=True)
    upd = jnp.concatenate([v, jnp.zeros((7, D), jnp.float32)], axis=0)

    @pl.when(i == 0)
    def _():
        sq_ref[...] = jnp.zeros_like(sq_ref)

    sq_ref[...] += upd


def _tc_var(m2, stats):
    return pl.pallas_call(
        _tc_var_body,
        grid=(NBLK,),
        in_specs=[
            pl.BlockSpec((BR, D), lambda i: (i, 0)),
            pl.BlockSpec((8, D), lambda i: (0, 0)),
        ],
        out_specs=pl.BlockSpec((8, D), lambda i: (0, 0)),
        out_shape=jax.ShapeDtypeStruct((8, D), jnp.float32),
    )(m2, stats)


def _tc_norm_body(m2_ref, stats_ref, sq_ref, g_ref, be_ref, h_ref):
    mean = stats_ref[0:1, :] / N
    var = sq_ref[0:1, :] / N
    h_ref[...] = _leaky((m2_ref[...] - mean) / jnp.sqrt(var + 1e-5)
                        * g_ref[...] + be_ref[...])


def _tc_norm(m2, stats, sq, g, be):
    return pl.pallas_call(
        _tc_norm_body,
        grid=(NBLK,),
        in_specs=[
            pl.BlockSpec((BR, D), lambda i: (i, 0)),
            pl.BlockSpec((8, D), lambda i: (0, 0)),
            pl.BlockSpec((8, D), lambda i: (0, 0)),
            pl.BlockSpec((1, D), lambda i: (0, 0)),
            pl.BlockSpec((1, D), lambda i: (0, 0)),
        ],
        out_specs=pl.BlockSpec((BR, D), lambda i: (i, 0)),
        out_shape=jax.ShapeDtypeStruct((N, D), jnp.float32),
    )(m2, stats, sq, g, be)


def _tc_pool_body(m2_ref, stats_ref, sq_ref, g_ref, be_ref, batch_ref,
                  wf1_ref, bf1_ref, gf_ref, bef_ref, wf2_ref, bf2_ref,
                  out_ref, psum_ref, cnt_ref):
    i = pl.program_id(0)
    mean = stats_ref[0:1, :] / N
    var = sq_ref[0:1, :] / N
    h = _leaky((m2_ref[...] - mean) / jnp.sqrt(var + 1e-5)
               * g_ref[...] + be_ref[...])

    b = batch_ref[0, 0, :]
    seg = lax.broadcasted_iota(jnp.int32, (BR, B), 1)
    onehot = (b[:, None] == seg).astype(jnp.float32)

    @pl.when(i == 0)
    def _():
        psum_ref[...] = jnp.zeros_like(psum_ref)
        cnt_ref[...] = jnp.zeros_like(cnt_ref)

    dn = (((0,), (0,)), ((), ()))
    psum_ref[...] += lax.dot_general(onehot, h, dn,
                                     preferred_element_type=jnp.float32,
                                     precision=lax.Precision.HIGHEST)
    cnt_ref[...] += lax.dot_general(onehot, jnp.ones((BR, D), jnp.float32),
                                    dn, preferred_element_type=jnp.float32,
                                    precision=lax.Precision.HIGHEST)

    @pl.when(i == NBLK - 1)
    def _():
        pooled = psum_ref[...] / jnp.maximum(cnt_ref[...], 1.0)
        z = jnp.dot(pooled, wf1_ref[...], preferred_element_type=jnp.float32,
                    precision=lax.Precision.DEFAULT) + bf1_ref[...]
        zm = jnp.mean(z, axis=0, keepdims=True)
        zd = z - zm
        zv = jnp.mean(zd * zd, axis=0, keepdims=True)
        z = _leaky(zd / jnp.sqrt(zv + 1e-5) * gf_ref[...] + bef_ref[...])
        out_ref[...] = jnp.dot(z, wf2_ref[...], preferred_element_type=jnp.float32,
                               precision=lax.Precision.DEFAULT) + bf2_ref[...]


def _tc_pool(m2, stats, sq, g, be, batchr, wf1, bf1, gf, bef, wf2p, bf2p):
    return pl.pallas_call(
        _tc_pool_body,
        grid=(NBLK,),
        in_specs=[
            pl.BlockSpec((BR, D), lambda i: (i, 0)),
            pl.BlockSpec((8, D), lambda i: (0, 0)),
            pl.BlockSpec((8, D), lambda i: (0, 0)),
            pl.BlockSpec((1, D), lambda i: (0, 0)),
            pl.BlockSpec((1, D), lambda i: (0, 0)),
            pl.BlockSpec((1, 1, BR), lambda i: (i, 0, 0)),
            pl.BlockSpec((D, D), lambda i: (0, 0)),
            pl.BlockSpec((1, D), lambda i: (0, 0)),
            pl.BlockSpec((1, D), lambda i: (0, 0)),
            pl.BlockSpec((1, D), lambda i: (0, 0)),
            pl.BlockSpec((D, 16), lambda i: (0, 0)),
            pl.BlockSpec((1, 16), lambda i: (0, 0)),
        ],
        out_specs=pl.BlockSpec((B, 16), lambda i: (0, 0)),
        out_shape=jax.ShapeDtypeStruct((B, 16), jnp.float32),
        scratch_shapes=[
            pltpu.VMEM((B, D), jnp.float32),
            pltpu.VMEM((B, D), jnp.float32),
        ],
    )(m2, stats, sq, g, be, batchr, wf1, bf1, gf, bef, wf2p, bf2p)


def _prep_edges(src, dst):
    perm = jnp.argsort(dst, stable=True)
    src_s = src[perm]
    dst_s = dst[perm]
    starts = jnp.searchsorted(dst_s, jnp.arange(N + 1, dtype=jnp.int32))
    starts = starts.astype(jnp.int32)
    ps = jnp.sort(jnp.concatenate([
        starts, jnp.arange(1, NW, dtype=jnp.int32) * (E // NW)]))
    NPR = ps.shape[0] - 1
    NCOL = NW * K
    CL = E // NCOL
    colr = ps[:-1] // CL
    first_r = jnp.searchsorted(colr, jnp.arange(NCOL, dtype=jnp.int32),
                               side="left").astype(jnp.int32)
    s0col = ps[jnp.clip(first_r, 0, NPR - 1)]

    ei = jnp.arange(E, dtype=jnp.int32)
    pr_e = jnp.searchsorted(ps, ei, side="right").astype(jnp.int32) - 1
    cidx_e = ei - s0col[colr[pr_e]]
    total_ok = jnp.max(cidx_e) < NCHUNK

    slot = jnp.arange(EP, dtype=jnp.int32)
    t = slot // EPW
    rem = slot % EPW
    c = rem // K
    j = rem % K
    col = t * K + j
    e = s0col[col] + c
    ec = jnp.clip(e, 0, E - 1)
    rowe = dst_s[ec]
    pr_slot = jnp.searchsorted(ps, ec, side="right").astype(jnp.int32) - 1
    valid = (e < E) & (colr[pr_slot] == col)
    pad_dst = (N + (slot % (N_PAD - N))).astype(jnp.int32)
    pad_src = slot % N
    src_p = jnp.where(valid, src_s[ec], pad_src)
    dst_p = jnp.where(valid, rowe, pad_dst)
    tail = slot < E
    src_f = jnp.where(tail, src_s[jnp.clip(slot, 0, E - 1)], pad_src)
    dst_f = jnp.where(tail, dst_s[jnp.clip(slot, 0, E - 1)], pad_dst)
    srcr = jnp.where(total_ok, src_p, src_f).reshape(NW, NG, GC, K)
    dstr = jnp.where(total_ok, dst_p, dst_f).reshape(NW, NG, GC, K)
    return srcr, dstr


def kernel(x, edge_index, batch, node_ids,
           W1a, b1a, W1b, b1b, g1, be1,
           W2a, b2a, W2b, b2b, g2, be2,
           W3a, b3a, W3b, b3b, g3, be3,
           W4a, b4a, W4b, b4b, g4, be4,
           Wf1, bf1, gf, bef, Wf2, bf2):
    srcr, dstr = _prep_edges(edge_index[0], edge_index[1])
    zeros = jnp.zeros((N_PAD, D), jnp.float32)
    batchr = batch.reshape(NBLK, 1, BR)

    r1 = lambda v: v.reshape(1, D)
    layers = [
        (W1a, r1(b1a), W1b, r1(b1b), r1(g1), r1(be1)),
        (W2a, r1(b2a), W2b, r1(b2b), r1(g2), r1(be2)),
        (W3a, r1(b3a), W3b, r1(b3b), r1(g3), r1(be3)),
        (W4a, r1(b4a), W4b, r1(b4b), r1(g4), r1(be4)),
    ]

    h = x
    m2 = stats = sq = None
    for li, (wa, ba, wb, bb, g, be) in enumerate(layers):
        p = _get_sc_scatter()(h, srcr, dstr, zeros)
        m2, stats = _tc_layer(h, p, wa, ba, wb, bb)
        sq = _tc_var(m2, stats)
        if li < 3:
            h = _tc_norm(m2, stats, sq, g, be)

    wf2p = jnp.pad(Wf2, ((0, 0), (0, 16 - C)))
    bf2p = jnp.pad(bf2, (0, 16 - C)).reshape(1, 16)
    z = _tc_pool(m2, stats, sq, layers[3][4], layers[3][5], batchr,
                 Wf1, r1(bf1), gf.reshape(1, D), bef.reshape(1, D),
                 wf2p, bf2p)
    return z[:, :C]
